# Initial kernel scaffold; baseline (speedup 1.0000x reference)
import jax
import jax.numpy as jnp
from jax.experimental import pallas as pl


def kernel(data, global_embs, sense_embs, ctx_weight):
    raise NotImplementedError("write your pallas kernel here")


# speedup vs baseline: 7.7824x; 7.7824x over previous
import functools

import jax
import jax.numpy as jnp
from jax import lax
from jax.experimental import pallas as pl
from jax.experimental.pallas import tpu as pltpu
from jax.experimental.pallas import tpu_sc as plsc

VOCAB = 100000
SIZE = 64
BATCH = 16384
W2 = 10
NEG = 5
NSENSE = NEG + 1
NCOL = 22

NC = 2
NS = 16
NW = NC * NS
BPW = BATCH // NW
CHUNK = 64
NCHUNK = BPW // CHUNK
LANES = 16
QV = SIZE // LANES

CTX_COLS = tuple(range(W2))
SENSE_COLS = (11, 12, 13, 14, 15, 16)


def _sc_body(dataT_hbm, gtab_hbm, stab_hbm, cw_hbm, out_hbm,
             idxbuf, gbuf, sbuf, cwbuf, outbuf, sem):
    wid = lax.axis_index("s") * NC + lax.axis_index("c")
    base = wid * BPW

    pltpu.sync_copy(dataT_hbm.at[:, wid], idxbuf)
    pltpu.sync_copy(cw_hbm, cwbuf)

    cwv = [[cwbuf[w, pl.ds(q * LANES, LANES)] for q in range(QV)]
           for w in range(W2)]
    lane = lax.broadcasted_iota(jnp.int32, (LANES,), 0)

    for c in range(NCHUNK):
        copies = []
        for k, col in enumerate(CTX_COLS):
            copies.append(pltpu.async_copy(
                gtab_hbm.at[idxbuf.at[col, c]], gbuf.at[k], sem))
        for k, col in enumerate(SENSE_COLS):
            copies.append(pltpu.async_copy(
                stab_hbm.at[idxbuf.at[col, c]], sbuf.at[k], sem))
        for cp in copies:
            cp.wait()

        def body(b, ipvecs):
            bi = b & (LANES - 1)
            sel = lane == bi
            acc = []
            for q in range(QV):
                a = gbuf[0, b, pl.ds(q * LANES, LANES)] * cwv[0][q]
                for w in range(1, W2):
                    a = a + gbuf[w, b, pl.ds(q * LANES, LANES)] * cwv[w][q]
                acc.append(a)
            new = []
            for j in range(NSENSE):
                p = sbuf[j, b, pl.ds(0, LANES)] * acc[0]
                for q in range(1, QV):
                    p = p + sbuf[j, b, pl.ds(q * LANES, LANES)] * acc[q]
                ip = plsc.cumsum(p)[LANES - 1]
                new.append(jnp.where(sel, ip, ipvecs[j]))

            @pl.when(bi == LANES - 1)
            def _store():
                g0 = pl.multiple_of(b - (LANES - 1), LANES)
                for j in range(NSENSE):
                    outbuf[j, pl.ds(g0, LANES)] = new[j]

            return tuple(new)

        lax.fori_loop(0, CHUNK, body,
                      tuple(jnp.zeros((LANES,), jnp.float32)
                            for _ in range(NSENSE)),
                      unroll=False)
        pltpu.sync_copy(outbuf, out_hbm.at[:, pl.ds(base + c * CHUNK, CHUNK)])


_sc_ips = functools.partial(
    pl.kernel,
    out_type=jax.ShapeDtypeStruct((NSENSE, BATCH), jnp.float32),
    mesh=plsc.VectorSubcoreMesh(core_axis_name="c", subcore_axis_name="s"),
    compiler_params=pltpu.CompilerParams(
        needs_layout_passes=False, use_tc_tiling_on_sc=False),
    scratch_types=[
        pltpu.VMEM((NCOL, NCHUNK, CHUNK), jnp.int32),
        pltpu.VMEM((W2, CHUNK, SIZE), jnp.float32),
        pltpu.VMEM((NSENSE, CHUNK, SIZE), jnp.float32),
        pltpu.VMEM((W2, SIZE), jnp.float32),
        pltpu.VMEM((NSENSE, CHUNK), jnp.float32),
        pltpu.SemaphoreType.DMA,
    ],
)(_sc_body)


def _tc_loss_body(y_ref, m_ref, o_ref):
    y = y_ref[...]
    m = m_ref[...]
    pos = jnp.clip(y[0:1, :], -10.0, 10.0)
    neg = jnp.clip(y[1:NSENSE, :], -10.0, 10.0)
    pos_loss = jnp.sum(jnp.log1p(jnp.exp(-pos)), keepdims=True)
    neg_loss = jnp.sum(m * jnp.log1p(jnp.exp(neg)), keepdims=True)
    o_ref[...] = pos_loss + neg_loss


def kernel(data, global_embs, sense_embs, ctx_weight):
    dataT = data.T.reshape(NCOL, NW, NCHUNK, CHUNK)
    maskf = data[:, W2 + 2 + NEG:].astype(jnp.float32).T

    ips = _sc_ips(dataT, global_embs, sense_embs, ctx_weight)

    out = pl.pallas_call(
        _tc_loss_body,
        out_shape=jax.ShapeDtypeStruct((1, 1), jnp.float32),
    )(ips, maskf)
    return out[0, 0]
